# baseline (speedup 1.0000x reference)
import jax
import jax.numpy as jnp
from jax import lax
from jax.experimental import pallas as pl
from jax.experimental.pallas import tpu as pltpu

TOKENS = 32768
HIDDEN = 2048
FEAT_LEN = 1024
NUM_FEATS = 4
COPY_BLOCK = 512
FB = 128
NTILE = FEAT_LEN // FB + 1


def _plain_copy_body(emb_ref, out_ref):
    out_ref[...] = emb_ref[...]


CB = 256
NGAP = NUM_FEATS + 1


def _gap_copy_body(locs_s, emb_hbm, out_hbm):
    def pipe_body(emb_blk, out_blk):
        out_blk[...] = emb_blk[...]

    reach = locs_s[3] + FEAT_LEN
    reaches = [None] * NUM_FEATS
    reaches[3] = reach
    for i in (2, 1, 0):
        e = locs_s[i] + FEAT_LEN
        reaches[i] = jnp.where(locs_s[i + 1] <= e, jnp.maximum(e, reaches[i + 1]), e)

    for g in range(NGAP):
        gs = jnp.int32(0) if g == 0 else reaches[g - 1]
        ge = locs_s[g] if g < NUM_FEATS else jnp.int32(TOKENS)
        base_blk = jnp.minimum(gs // CB, TOKENS // CB - 1)
        n_blk = jnp.maximum((ge + CB - 1) // CB - base_blk, 0)
        pltpu.emit_pipeline(
            pipe_body,
            grid=(n_blk,),
            in_specs=[
                pl.BlockSpec((CB, HIDDEN), lambda c, b=base_blk: (b + c, 0))
            ],
            out_specs=[
                pl.BlockSpec((CB, HIDDEN), lambda c, b=base_blk: (b + c, 0))
            ],
        )(emb_hbm, out_hbm)


def _feat_body(i, locs_s, fb_ref, cur_hbm, out_ref, prev, e0, e1):
    t = pl.program_id(0)
    loc = locs_s[i]
    base = pl.multiple_of((loc // FB) * FB, FB)
    r = loc - base

    @pl.when(t == 0)
    def _():
        pltpu.sync_copy(cur_hbm.at[pl.ds(base, FB)], e0)
        pltpu.sync_copy(cur_hbm.at[pl.ds(base + FEAT_LEN, FB)], e1)

    fb = fb_ref[...]
    fa = jnp.where(t == 0, fb, prev[...])
    u = jnp.concatenate([fa, fb], axis=0)
    v = pltpu.roll(u, FB + r, 0)[:FB, :]
    g = base + FB * t + lax.broadcasted_iota(jnp.int32, (FB, 1), 0)
    infeat = (g >= loc) & (g < loc + FEAT_LEN)
    cur_tile = jnp.where(t == 0, e0[...], e1[...])
    out_ref[...] = jnp.where(infeat, v, cur_tile)
    prev[...] = fb


def kernel(embeddings, feature_0, feature_1, feature_2, feature_3, multimodal_locs):
    locs = multimodal_locs.astype(jnp.int32)

    out = pl.pallas_call(
        _gap_copy_body,
        in_specs=[
            pl.BlockSpec(memory_space=pltpu.SMEM),
            pl.BlockSpec(memory_space=pltpu.MemorySpace.HBM),
        ],
        out_specs=pl.BlockSpec(memory_space=pltpu.MemorySpace.HBM),
        out_shape=jax.ShapeDtypeStruct((TOKENS, HIDDEN), jnp.float32),
    )(locs, embeddings)

    feats = [feature_0, feature_1, feature_2, feature_3]
    for i in range(NUM_FEATS):
        grid_spec = pltpu.PrefetchScalarGridSpec(
            num_scalar_prefetch=1,
            grid=(NTILE,),
            in_specs=[
                pl.BlockSpec(
                    (FB, HIDDEN),
                    lambda t, locs_ref: (jnp.minimum(t, FEAT_LEN // FB - 1), 0),
                ),
                pl.BlockSpec(memory_space=pltpu.MemorySpace.HBM),
            ],
            out_specs=pl.BlockSpec(
                (FB, HIDDEN),
                lambda t, locs_ref, i=i: (locs_ref[i] // FB + t, 0),
            ),
            scratch_shapes=[
                pltpu.VMEM((FB, HIDDEN), jnp.float32),
                pltpu.VMEM((FB, HIDDEN), jnp.float32),
                pltpu.VMEM((FB, HIDDEN), jnp.float32),
            ],
        )
        out = pl.pallas_call(
            lambda *a, i=i: _feat_body(i, *a),
            grid_spec=grid_spec,
            out_shape=jax.ShapeDtypeStruct((TOKENS, HIDDEN), jnp.float32),
            input_output_aliases={2: 0},
        )(locs, feats[i], out)
    return out


# speedup vs baseline: 1.0390x; 1.0390x over previous
import jax
import jax.numpy as jnp
from jax import lax
from jax.experimental import pallas as pl
from jax.experimental.pallas import tpu as pltpu

TOKENS = 32768
HIDDEN = 2048
FEAT_LEN = 1024
NUM_FEATS = 4
COPY_BLOCK = 1024
FB = 128
NTILE = FEAT_LEN // FB + 1


def _plain_copy_body(emb_ref, out_ref):
    out_ref[...] = emb_ref[...]


def _feat_body(i, locs_s, fb_ref, cur_hbm, out_ref, prev, e0, e1):
    t = pl.program_id(0)
    loc = locs_s[i]
    base = pl.multiple_of((loc // FB) * FB, FB)
    r = loc - base

    @pl.when(t == 0)
    def _():
        pltpu.sync_copy(cur_hbm.at[pl.ds(base, FB)], e0)
        pltpu.sync_copy(cur_hbm.at[pl.ds(base + FEAT_LEN, FB)], e1)

    fb = fb_ref[...]
    fa = jnp.where(t == 0, fb, prev[...])
    u = jnp.concatenate([fa, fb], axis=0)
    v = pltpu.roll(u, FB + r, 0)[:FB, :]
    g = base + FB * t + lax.broadcasted_iota(jnp.int32, (FB, 1), 0)
    infeat = (g >= loc) & (g < loc + FEAT_LEN)
    cur_tile = jnp.where(t == 0, e0[...], e1[...])
    out_ref[...] = jnp.where(infeat, v, cur_tile)
    prev[...] = fb


def kernel(embeddings, feature_0, feature_1, feature_2, feature_3, multimodal_locs):
    locs = multimodal_locs.astype(jnp.int32)

    out = pl.pallas_call(
        _plain_copy_body,
        grid=(TOKENS // COPY_BLOCK,),
        in_specs=[pl.BlockSpec((COPY_BLOCK, HIDDEN), lambda c: (c, 0))],
        out_specs=pl.BlockSpec((COPY_BLOCK, HIDDEN), lambda c: (c, 0)),
        out_shape=jax.ShapeDtypeStruct((TOKENS, HIDDEN), jnp.float32),
    )(embeddings)

    feats = [feature_0, feature_1, feature_2, feature_3]
    for i in range(NUM_FEATS):
        grid_spec = pltpu.PrefetchScalarGridSpec(
            num_scalar_prefetch=1,
            grid=(NTILE,),
            in_specs=[
                pl.BlockSpec(
                    (FB, HIDDEN),
                    lambda t, locs_ref: (jnp.minimum(t, FEAT_LEN // FB - 1), 0),
                ),
                pl.BlockSpec(memory_space=pltpu.MemorySpace.HBM),
            ],
            out_specs=pl.BlockSpec(
                (FB, HIDDEN),
                lambda t, locs_ref, i=i: (locs_ref[i] // FB + t, 0),
            ),
            scratch_shapes=[
                pltpu.VMEM((FB, HIDDEN), jnp.float32),
                pltpu.VMEM((FB, HIDDEN), jnp.float32),
                pltpu.VMEM((FB, HIDDEN), jnp.float32),
            ],
        )
        out = pl.pallas_call(
            lambda *a, i=i: _feat_body(i, *a),
            grid_spec=grid_spec,
            out_shape=jax.ShapeDtypeStruct((TOKENS, HIDDEN), jnp.float32),
            input_output_aliases={2: 0},
        )(locs, feats[i], out)
    return out
